# baseline (speedup 1.0000x reference)
import functools

import jax
import jax.numpy as jnp
from jax import lax
from jax.experimental import pallas as pl
from jax.experimental.pallas import tpu as pltpu
from jax.experimental.pallas import tpu_sc as plsc

N = 10000
E = 160000
D_IN = 128
H = 4
C = 64
G = 64
HC = H * C
HHC = HC // 2
NC = 2
NS = 16
L = 16
EPW = E // NS
CH = 80
NCHUNK = EPW // CH
IB = 5
IBCH = IB * CH
NPAD = 10240
NPW = NPAD // NS
DR = NPAD // 16
BE = 2000


def _tc_proj_kernel(x_ref, gamma_ref, beta_ref, wl_ref, bl_ref, wr_ref,
                    br_ref, xlt_ref, xrt_ref):
    x = x_ref[...]
    mean = jnp.mean(x, axis=0, keepdims=True)
    var = jnp.mean((x - mean) ** 2, axis=0, keepdims=True)
    xn = (x - mean) * lax.rsqrt(var + 1e-5) * gamma_ref[...] + beta_ref[...]
    xl = jnp.dot(xn, wl_ref[...], preferred_element_type=jnp.float32) + bl_ref[...]
    xr = jnp.dot(xn, wr_ref[...], preferred_element_type=jnp.float32) + br_ref[...]
    xlt_ref[0] = xl[:, :HHC]
    xlt_ref[1] = xl[:, HHC:]
    xrt_ref[0] = xr[:, :HHC]
    xrt_ref[1] = xr[:, HHC:]


def _tc_ee_kernel(ea_ref, we_ref, ee_ref):
    ee = jnp.dot(ea_ref[...], we_ref[...], preferred_element_type=jnp.float32)
    ee_ref[0] = ee[:, :HHC]
    ee_ref[1] = ee[:, HHC:]


def _sc_edge_kernel(src_hbm, dst_hbm, ee_hbm, xlt_hbm, xrt_hbm, att_hbm,
                    agg_hbm, den_hbm,
                    srcb, dstb, srcpb, dstpb, dnidx, dstc, xlb, xrb, eeb,
                    denb, attv, agg_sh, den_sh, gsem1, gsem2, esem,
                    ssem1, ssem2):
    c = lax.axis_index("c")
    s = lax.axis_index("s")

    zero = jnp.zeros((L,), jnp.float32)

    def zrow(i, carry):
        for j in range(HHC // L):
            xlb[i, pl.ds(L * j, L)] = zero
            denb[i, pl.ds(L * j, L)] = zero
        return carry

    lax.fori_loop(0, CH, zrow, 0)
    for t in range(NPW // CH):
        pltpu.sync_copy(xlb, agg_sh.at[pl.ds(s * NPW + t * CH, CH)])
    pltpu.sync_copy(denb.at[pl.ds(0, DR // NS)],
                    den_sh.at[pl.ds(s * (DR // NS), DR // NS)])
    plsc.subcore_barrier()

    pltpu.sync_copy(att_hbm.at[pl.ds(c * HHC, HHC)], attv)
    atts = [attv[pl.ds(L * j, L)] for j in range(HHC // L)]
    lane = lax.iota(jnp.int32, L)
    perms = [lane ^ sh for sh in (8, 4, 2, 1)]
    bc0 = lane * 0
    nbase = c * N

    def hsum_bcast(v):
        for p in perms:
            v = v + v.at[p].get(mode="promise_in_bounds")
        return v

    def super_body(q, carry):
        eq = s * EPW + q * IBCH
        pltpu.sync_copy(src_hbm.at[pl.ds(eq, IBCH)], srcb)
        pltpu.sync_copy(dst_hbm.at[pl.ds(eq, IBCH)], dstb)
        prev = None
        for b in range(IB):
            prev = _chunk(c, eq + b * CH, b, srcb, dstb, srcpb, dstpb,
                          dnidx.at[b % 2], dstc.at[b % 2], xlb, xrb, eeb,
                          denb, atts, lane, perms, bc0, nbase, hsum_bcast,
                          ee_hbm, xlt_hbm, xrt_hbm, agg_sh, den_sh,
                          gsem1, gsem2, esem, ssem1, ssem2,
                          prev, b == IB - 1)
        return carry

    lax.fori_loop(0, NCHUNK // IB, super_body, 0)
    plsc.subcore_barrier()

    row0 = c * NPAD + s * NPW
    pltpu.sync_copy(agg_sh.at[pl.ds(s * NPW, NPW)], agg_hbm.at[pl.ds(row0, NPW)])
    pltpu.sync_copy(den_sh.at[pl.ds(s * (DR // NS), DR // NS)],
                    den_hbm.at[pl.ds(c * DR + s * (DR // NS), DR // NS)])


def _chunk(c, e0, b, srcb, dstb, srcpb, dstpb, dnidx, dstc, xlb, xrb,
           eeb, denb, atts, lane, perms, bc0, nbase, hsum_bcast,
           ee_hbm, xlt_hbm, xrt_hbm, agg_sh, den_sh, gsem1, gsem2, esem,
           ssem1, ssem2, prev, last):
    if True:
        for j in range(CH // L):
            o = b * CH + L * j
            srcpb[pl.ds(L * j, L)] = srcb[pl.ds(o, L)] + nbase
            d = dstb[pl.ds(o, L)]
            dstc[pl.ds(L * j, L)] = d
            dstpb[pl.ds(L * j, L)] = d + nbase
            dnidx[pl.ds(L * j, L)] = lax.shift_right_logical(d, 4)
        cp2 = pltpu.async_copy(xrt_hbm.at[dstpb], xrb, gsem2)
        cpe = pltpu.async_copy(ee_hbm.at[pl.ds(c * E + e0, CH)], eeb, esem)
        if prev is not None:
            prev[0].wait()
            prev[1].wait()
        cp1 = pltpu.async_copy(xlt_hbm.at[srcpb], xlb, gsem1)
        cpe.wait()
        cp1.wait()
        cp2.wait()

        def group_body(g, gcarry):
            dd = dstc[pl.ds(g * L, L)]

            def do_edge(i, t):
                xls = [xlb[i, pl.ds(L * j, L)] for j in range(HHC // L)]
                p0 = None
                p1 = None
                for j in range(HHC // L):
                    v = (xls[j] + xrb[i, pl.ds(L * j, L)]
                         + eeb[i, pl.ds(L * j, L)])
                    w = jnp.maximum(v, 0.2 * v)
                    u = w * atts[j]
                    if j < 4:
                        p0 = u if p0 is None else p0 + u
                    else:
                        p1 = u if p1 is None else p1 + u
                ex0 = jnp.exp(hsum_bcast(p0))
                ex1 = jnp.exp(hsum_bcast(p1))
                for j in range(4):
                    xlb[i, pl.ds(L * j, L)] = ex0 * xls[j]
                for j in range(4, 8):
                    xlb[i, pl.ds(L * j, L)] = ex1 * xls[j]
                dvec = dd.at[bc0 + t].get(mode="promise_in_bounds")
                m = lane == (dvec & 15)
                denb[i, pl.ds(0, L)] = jnp.where(m, ex0, 0.0)
                denb[i, pl.ds(L, L)] = jnp.where(m, ex1, 0.0)

            def edge_body(tt, ecarry):
                t = 2 * tt
                do_edge(g * L + t, t)
                do_edge(g * L + t + 1, t + 1)
                return ecarry

            return lax.fori_loop(0, L // 2, edge_body, gcarry)

        lax.fori_loop(0, CH // L, group_body, 0)
        if last:
            pltpu.sync_copy(xlb, agg_sh.at[dstc], add=True)
            pltpu.sync_copy(denb, den_sh.at[dnidx], add=True)
            return None
        h1 = pltpu.async_copy(xlb, agg_sh.at[dstc], ssem1, add=True)
        h2 = pltpu.async_copy(denb, den_sh.at[dnidx], ssem2, add=True)
        return (h1, h2)


_sc_edge = functools.partial(
    pl.kernel,
    out_type=(
        jax.ShapeDtypeStruct((2 * NPAD, HHC), jnp.float32),
        jax.ShapeDtypeStruct((2 * DR, HHC), jnp.float32),
    ),
    mesh=plsc.VectorSubcoreMesh(
        core_axis_name="c", subcore_axis_name="s",
        num_cores=NC, num_subcores=NS),
    scratch_types=[
        pltpu.VMEM((IBCH,), jnp.int32),
        pltpu.VMEM((IBCH,), jnp.int32),
        pltpu.VMEM((CH,), jnp.int32),
        pltpu.VMEM((CH,), jnp.int32),
        pltpu.VMEM((2, CH), jnp.int32),
        pltpu.VMEM((2, CH), jnp.int32),
        pltpu.VMEM((CH, HHC), jnp.float32),
        pltpu.VMEM((CH, HHC), jnp.float32),
        pltpu.VMEM((CH, HHC), jnp.float32),
        pltpu.VMEM((CH, HHC), jnp.float32),
        pltpu.VMEM((HHC,), jnp.float32),
        pltpu.VMEM_SHARED((NPAD, HHC), jnp.float32),
        pltpu.VMEM_SHARED((DR, HHC), jnp.float32),
        pltpu.SemaphoreType.DMA,
        pltpu.SemaphoreType.DMA,
        pltpu.SemaphoreType.DMA,
        pltpu.SemaphoreType.DMA,
        pltpu.SemaphoreType.DMA,
    ],
)(_sc_edge_kernel)


def _tc_final_kernel(agg_ref, den_ref, batch_ref, bias_ref, wfin_ref,
                     bfin_ref, out_ref):
    eps = 1e-16
    a = agg_ref[...]
    dn = den_ref[...]
    hsum = (a[:N, :C] / (dn[:, 0:1] + eps)
            + a[:N, C:] / (dn[:, 1:2] + eps)
            + a[NPAD:NPAD + N, :C] / (dn[:, 2:3] + eps)
            + a[NPAD:NPAD + N, C:] / (dn[:, 3:4] + eps))
    h = jnp.maximum(0.25 * hsum + bias_ref[...], 0.0)
    g = lax.broadcasted_iota(jnp.int32, (N, G), 1)
    oh = (batch_ref[...] == g).astype(jnp.float32)
    pooled_sum = lax.dot_general(oh, h, (((0,), (0,)), ((), ())),
                                 preferred_element_type=jnp.float32)
    counts = jnp.sum(oh, axis=0)[:, None]
    pooled = pooled_sum / jnp.maximum(counts, 1.0)
    out_ref[...] = (jnp.dot(pooled, wfin_ref[...],
                            preferred_element_type=jnp.float32)
                    + bfin_ref[...])


def kernel(x, edge_index, edge_attr, batch, gamma, beta, W_l, b_l, W_r, b_r,
           W_e, att, bias_out, W_fin, b_fin):
    xlt, xrt = pl.pallas_call(
        _tc_proj_kernel,
        out_shape=(
            jax.ShapeDtypeStruct((2, N, HHC), jnp.float32),
            jax.ShapeDtypeStruct((2, N, HHC), jnp.float32),
        ),
    )(x, gamma.reshape(1, D_IN), beta.reshape(1, D_IN),
      W_l, b_l.reshape(1, HC), W_r, b_r.reshape(1, HC))

    eet = pl.pallas_call(
        _tc_ee_kernel,
        grid=(E // BE,),
        in_specs=[
            pl.BlockSpec((BE, D_IN), lambda i: (i, 0)),
            pl.BlockSpec((D_IN, HC), lambda i: (0, 0)),
        ],
        out_specs=pl.BlockSpec((2, BE, HHC), lambda i: (0, i, 0)),
        out_shape=jax.ShapeDtypeStruct((2, E, HHC), jnp.float32),
    )(edge_attr, W_e)

    src = edge_index[0]
    dst = edge_index[1]
    att2 = att.reshape(HC)

    agg, den = _sc_edge(src, dst, eet.reshape(2 * E, HHC),
                        xlt.reshape(2 * N, HHC), xrt.reshape(2 * N, HHC),
                        att2)

    dr = den.reshape(2, DR, HHC)
    e0 = dr[:, :, :L].reshape(2, DR * L)[:, :N]
    e1 = dr[:, :, L:2 * L].reshape(2, DR * L)[:, :N]
    den_nodes = jnp.stack([e0[0], e1[0], e0[1], e1[1]], axis=1)

    wfin_pad = jnp.pad(W_fin, ((0, 0), (0, HHC - 2)))
    bfin_pad = jnp.pad(b_fin, (0, HHC - 2)).reshape(1, HHC)
    outp = pl.pallas_call(
        _tc_final_kernel,
        out_shape=jax.ShapeDtypeStruct((G, HHC), jnp.float32),
    )(agg, den_nodes, batch.reshape(N, 1), bias_out.reshape(1, C),
      wfin_pad, bfin_pad)
    return outp[:, :2]


# speedup vs baseline: 1.0277x; 1.0277x over previous
import functools

import jax
import jax.numpy as jnp
from jax import lax
from jax.experimental import pallas as pl
from jax.experimental.pallas import tpu as pltpu
from jax.experimental.pallas import tpu_sc as plsc

N = 10000
E = 160000
D_IN = 128
H = 4
C = 64
G = 64
HC = H * C
HHC = HC // 2
NC = 2
NS = 16
L = 16
EPW = E // NS
CH = 80
NCHUNK = EPW // CH
IB = 5
IBCH = IB * CH
NPAD = 10240
NPW = NPAD // NS
DR = NPAD // 16
BE = 2000


def _tc_proj_kernel(x_ref, gamma_ref, beta_ref, wl_ref, bl_ref, wr_ref,
                    br_ref, xlt_ref, xrt_ref):
    x = x_ref[...]
    mean = jnp.mean(x, axis=0, keepdims=True)
    var = jnp.mean((x - mean) ** 2, axis=0, keepdims=True)
    xn = (x - mean) * lax.rsqrt(var + 1e-5) * gamma_ref[...] + beta_ref[...]
    xl = jnp.dot(xn, wl_ref[...], preferred_element_type=jnp.float32) + bl_ref[...]
    xr = jnp.dot(xn, wr_ref[...], preferred_element_type=jnp.float32) + br_ref[...]
    xlt_ref[0] = xl[:, :HHC]
    xlt_ref[1] = xl[:, HHC:]
    xrt_ref[0] = xr[:, :HHC]
    xrt_ref[1] = xr[:, HHC:]


def _tc_ee_kernel(ea_ref, we_ref, ee_ref):
    ee = jnp.dot(ea_ref[...], we_ref[...], preferred_element_type=jnp.float32)
    ee_ref[0] = ee[:, :HHC]
    ee_ref[1] = ee[:, HHC:]


def _sc_edge_kernel(src_hbm, dst_hbm, ee_hbm, xlt_hbm, xrt_hbm, att_hbm,
                    agg_hbm, den_hbm,
                    srcb, dstb, srcpb, dstpb, dnidx, dstc, xlb, xrb, eeb,
                    denb, attv, agg_sh, den_sh, gsem1, gsem2, esem,
                    ssem1, ssem2):
    c = lax.axis_index("c")
    s = lax.axis_index("s")

    zero = jnp.zeros((L,), jnp.float32)

    def zrow(i, carry):
        for j in range(HHC // L):
            xlb[i, pl.ds(L * j, L)] = zero
            denb[i, pl.ds(L * j, L)] = zero
        return carry

    lax.fori_loop(0, CH, zrow, 0)
    for t in range(NPW // CH):
        pltpu.sync_copy(xlb, agg_sh.at[pl.ds(s * NPW + t * CH, CH)])
    pltpu.sync_copy(denb.at[pl.ds(0, DR // NS)],
                    den_sh.at[pl.ds(s * (DR // NS), DR // NS)])
    plsc.subcore_barrier()

    pltpu.sync_copy(att_hbm.at[pl.ds(c * HHC, HHC)], attv)
    atts = [attv[pl.ds(L * j, L)] for j in range(HHC // L)]
    lane = lax.iota(jnp.int32, L)
    perms = [lane ^ sh for sh in (8, 4, 2, 1)]
    bc0 = lane * 0
    nbase = c * N

    def hsum_bcast(v):
        for p in perms:
            v = v + v.at[p].get(mode="promise_in_bounds")
        return v

    def super_body(q, carry):
        eq = s * EPW + q * IBCH
        pltpu.sync_copy(src_hbm.at[pl.ds(eq, IBCH)], srcb)
        pltpu.sync_copy(dst_hbm.at[pl.ds(eq, IBCH)], dstb)
        prev = None
        for b in range(IB):
            prev = _chunk(c, eq + b * CH, b, srcb, dstb, srcpb, dstpb,
                          dnidx.at[b % 2], dstc.at[b % 2], xlb, xrb, eeb,
                          denb, atts, lane, perms, bc0, nbase, hsum_bcast,
                          ee_hbm, xlt_hbm, xrt_hbm, agg_sh, den_sh,
                          gsem1, gsem2, esem, ssem1, ssem2,
                          prev, b == IB - 1)
        return carry

    lax.fori_loop(0, NCHUNK // IB, super_body, 0)
    plsc.subcore_barrier()

    row0 = c * NPAD + s * NPW
    pltpu.sync_copy(agg_sh.at[pl.ds(s * NPW, NPW)], agg_hbm.at[pl.ds(row0, NPW)])
    pltpu.sync_copy(den_sh.at[pl.ds(s * (DR // NS), DR // NS)],
                    den_hbm.at[pl.ds(c * DR + s * (DR // NS), DR // NS)])


def _chunk(c, e0, b, srcb, dstb, srcpb, dstpb, dnidx, dstc, xlb, xrb,
           eeb, denb, atts, lane, perms, bc0, nbase, hsum_bcast,
           ee_hbm, xlt_hbm, xrt_hbm, agg_sh, den_sh, gsem1, gsem2, esem,
           ssem1, ssem2, prev, last):
    if True:
        for j in range(CH // L):
            o = b * CH + L * j
            srcpb[pl.ds(L * j, L)] = srcb[pl.ds(o, L)] + nbase
            d = dstb[pl.ds(o, L)]
            dstc[pl.ds(L * j, L)] = d
            dstpb[pl.ds(L * j, L)] = d + nbase
            dnidx[pl.ds(L * j, L)] = lax.shift_right_logical(d, 4)
        cp2 = pltpu.async_copy(xrt_hbm.at[dstpb], xrb, gsem2)
        cpe = pltpu.async_copy(ee_hbm.at[pl.ds(c * E + e0, CH)], eeb, esem)
        if prev is not None:
            prev[0].wait()
            prev[1].wait()
        cp1 = pltpu.async_copy(xlt_hbm.at[srcpb], xlb, gsem1)
        cpe.wait()
        cp1.wait()
        cp2.wait()

        def group_body(g, gcarry):
            dd = dstc[pl.ds(g * L, L)]

            def edge_body(t, ecarry):
                i = g * L + t
                xls = [xlb[i, pl.ds(L * j, L)] for j in range(HHC // L)]
                p0 = None
                p1 = None
                for j in range(HHC // L):
                    v = (xls[j] + xrb[i, pl.ds(L * j, L)]
                         + eeb[i, pl.ds(L * j, L)])
                    w = jnp.maximum(v, 0.2 * v)
                    u = w * atts[j]
                    if j < 4:
                        p0 = u if p0 is None else p0 + u
                    else:
                        p1 = u if p1 is None else p1 + u
                ex0 = jnp.exp(hsum_bcast(p0))
                ex1 = jnp.exp(hsum_bcast(p1))
                for j in range(4):
                    xlb[i, pl.ds(L * j, L)] = ex0 * xls[j]
                for j in range(4, 8):
                    xlb[i, pl.ds(L * j, L)] = ex1 * xls[j]
                dvec = dd.at[bc0 + t].get(mode="promise_in_bounds")
                m = lane == (dvec & 15)
                denb[i, pl.ds(0, L)] = jnp.where(m, ex0, 0.0)
                denb[i, pl.ds(L, L)] = jnp.where(m, ex1, 0.0)
                return ecarry

            return lax.fori_loop(0, L, edge_body, gcarry)

        lax.fori_loop(0, CH // L, group_body, 0)
        if last:
            pltpu.sync_copy(xlb, agg_sh.at[dstc], add=True)
            pltpu.sync_copy(denb, den_sh.at[dnidx], add=True)
            return None
        h1 = pltpu.async_copy(xlb, agg_sh.at[dstc], ssem1, add=True)
        h2 = pltpu.async_copy(denb, den_sh.at[dnidx], ssem2, add=True)
        return (h1, h2)


_sc_edge = functools.partial(
    pl.kernel,
    out_type=(
        jax.ShapeDtypeStruct((2 * NPAD, HHC), jnp.float32),
        jax.ShapeDtypeStruct((2 * DR, HHC), jnp.float32),
    ),
    mesh=plsc.VectorSubcoreMesh(
        core_axis_name="c", subcore_axis_name="s",
        num_cores=NC, num_subcores=NS),
    scratch_types=[
        pltpu.VMEM((IBCH,), jnp.int32),
        pltpu.VMEM((IBCH,), jnp.int32),
        pltpu.VMEM((CH,), jnp.int32),
        pltpu.VMEM((CH,), jnp.int32),
        pltpu.VMEM((2, CH), jnp.int32),
        pltpu.VMEM((2, CH), jnp.int32),
        pltpu.VMEM((CH, HHC), jnp.float32),
        pltpu.VMEM((CH, HHC), jnp.float32),
        pltpu.VMEM((CH, HHC), jnp.float32),
        pltpu.VMEM((CH, HHC), jnp.float32),
        pltpu.VMEM((HHC,), jnp.float32),
        pltpu.VMEM_SHARED((NPAD, HHC), jnp.float32),
        pltpu.VMEM_SHARED((DR, HHC), jnp.float32),
        pltpu.SemaphoreType.DMA,
        pltpu.SemaphoreType.DMA,
        pltpu.SemaphoreType.DMA,
        pltpu.SemaphoreType.DMA,
        pltpu.SemaphoreType.DMA,
    ],
)(_sc_edge_kernel)


def _tc_final_kernel(agg_ref, den_ref, batch_ref, bias_ref, wfin_ref,
                     bfin_ref, out_ref):
    eps = 1e-16
    a = agg_ref[...]
    dn = den_ref[...]
    hsum = (a[:N, :C] / (dn[:, 0:1] + eps)
            + a[:N, C:] / (dn[:, 1:2] + eps)
            + a[NPAD:NPAD + N, :C] / (dn[:, 2:3] + eps)
            + a[NPAD:NPAD + N, C:] / (dn[:, 3:4] + eps))
    h = jnp.maximum(0.25 * hsum + bias_ref[...], 0.0)
    g = lax.broadcasted_iota(jnp.int32, (N, G), 1)
    oh = (batch_ref[...] == g).astype(jnp.float32)
    pooled_sum = lax.dot_general(oh, h, (((0,), (0,)), ((), ())),
                                 preferred_element_type=jnp.float32)
    counts = jnp.sum(oh, axis=0)[:, None]
    pooled = pooled_sum / jnp.maximum(counts, 1.0)
    out_ref[...] = (jnp.dot(pooled, wfin_ref[...],
                            preferred_element_type=jnp.float32)
                    + bfin_ref[...])


def kernel(x, edge_index, edge_attr, batch, gamma, beta, W_l, b_l, W_r, b_r,
           W_e, att, bias_out, W_fin, b_fin):
    xlt, xrt = pl.pallas_call(
        _tc_proj_kernel,
        out_shape=(
            jax.ShapeDtypeStruct((2, N, HHC), jnp.float32),
            jax.ShapeDtypeStruct((2, N, HHC), jnp.float32),
        ),
    )(x, gamma.reshape(1, D_IN), beta.reshape(1, D_IN),
      W_l, b_l.reshape(1, HC), W_r, b_r.reshape(1, HC))

    eet = pl.pallas_call(
        _tc_ee_kernel,
        grid=(E // BE,),
        in_specs=[
            pl.BlockSpec((BE, D_IN), lambda i: (i, 0)),
            pl.BlockSpec((D_IN, HC), lambda i: (0, 0)),
        ],
        out_specs=pl.BlockSpec((2, BE, HHC), lambda i: (0, i, 0)),
        out_shape=jax.ShapeDtypeStruct((2, E, HHC), jnp.float32),
    )(edge_attr, W_e)

    src = edge_index[0]
    dst = edge_index[1]
    att2 = att.reshape(HC)

    agg, den = _sc_edge(src, dst, eet.reshape(2 * E, HHC),
                        xlt.reshape(2 * N, HHC), xrt.reshape(2 * N, HHC),
                        att2)

    dr = den.reshape(2, DR, HHC)
    e0 = dr[:, :, :L].reshape(2, DR * L)[:, :N]
    e1 = dr[:, :, L:2 * L].reshape(2, DR * L)[:, :N]
    den_nodes = jnp.stack([e0[0], e1[0], e0[1], e1[1]], axis=1)

    wfin_pad = jnp.pad(W_fin, ((0, 0), (0, HHC - 2)))
    bfin_pad = jnp.pad(b_fin, (0, HHC - 2)).reshape(1, HHC)
    outp = pl.pallas_call(
        _tc_final_kernel,
        out_shape=jax.ShapeDtypeStruct((G, HHC), jnp.float32),
    )(agg, den_nodes, batch.reshape(N, 1), bias_out.reshape(1, C),
      wfin_pad, bfin_pad)
    return outp[:, :2]
